# baseline (speedup 1.0000x reference)
import jax
import jax.numpy as jnp
import numpy as np
from jax.experimental import pallas as pl
from jax.experimental.pallas import tpu as pltpu

_N_LAYERS = 4
_BAND = _N_LAYERS + 3


def _coeffs():
    es = [1.0] + [float(l + 3) for l in range(_N_LAYERS)] + [float(_N_LAYERS + 3)]
    out = []
    for e in es:
        dinv = np.float32(np.float32(e + 1.0) ** np.float32(-0.5))
        alpha = np.float32(np.float32(dinv * np.float32(e)) * dinv)
        beta = np.float32(dinv * dinv)
        out.append((alpha, beta))
    return out


_COEFFS = _coeffs()


def _body(datas_ref, tag_ref, emb_hbm, w0_hbm, b0_hbm, ws_hbm, bs_hbm,
          we_hbm, be_ref, wfc_hbm, bfc_ref, out_ref,
          x_scr, w0_s, b0_s, ws_s, bs_s, we_s, wfc_s, sem):
    nb, seq = datas_ref.shape
    copies = [
        pltpu.make_async_copy(
            emb_hbm.at[pl.ds(datas_ref[nb - 1, seq - _BAND + j], 1), :],
            x_scr.at[pl.ds(j, 1), :], sem)
        for j in range(_BAND)
    ] + [
        pltpu.make_async_copy(w0_hbm, w0_s, sem),
        pltpu.make_async_copy(b0_hbm, b0_s, sem),
        pltpu.make_async_copy(ws_hbm, ws_s, sem),
        pltpu.make_async_copy(bs_hbm, bs_s, sem),
        pltpu.make_async_copy(we_hbm, we_s, sem),
        pltpu.make_async_copy(wfc_hbm, wfc_s, sem),
    ]
    for c in copies:
        c.start()
    x_scr[pl.ds(_BAND, 1), :] = jnp.zeros((1, 128), jnp.float32)
    for c in copies:
        c.wait()

    def conv(x, w, b, k, relu):
        a, bt = _COEFFS[k]
        y = jnp.dot(x, w, preferred_element_type=jnp.float32)
        shifted = jnp.concatenate([jnp.zeros_like(y[:1]), y[:-1]], axis=0)
        y = a * shifted + bt * y + b
        return jnp.maximum(y, 0.0) if relu else y

    x = x_scr[...]
    x = conv(x, w0_s[...], b0_s[...], 0, True)
    for l in range(_N_LAYERS):
        x = conv(x, ws_s[l], bs_s[l:l + 1, :], l + 1, True)
    x = conv(x, we_s[...], be_ref[...], _N_LAYERS + 1, False)
    pre = jnp.dot(x, wfc_s[...], preferred_element_type=jnp.float32)
    pre = pre + bfc_ref[...]
    row = pre[_BAND - 1:_BAND, :]
    m = jnp.max(row, axis=1, keepdims=True)
    lse = m + jnp.log(jnp.sum(jnp.exp(row - m), axis=1, keepdims=True))
    lane = jax.lax.broadcasted_iota(jnp.int32, row.shape, 1)
    picked = jnp.sum(jnp.where(lane == tag_ref[0], row, 0.0), axis=1,
                     keepdims=True)
    out_ref[...] = lse - picked


def kernel(batch_datas, batch_tags, emb_table, W_start, b_start, Ws, bs,
           W_end, b_end, W_fc, b_fc):
    grid_spec = pltpu.PrefetchScalarGridSpec(
        num_scalar_prefetch=2,
        grid=(1,),
        in_specs=[
            pl.BlockSpec(memory_space=pl.ANY),
            pl.BlockSpec(memory_space=pl.ANY),
            pl.BlockSpec(memory_space=pl.ANY),
            pl.BlockSpec(memory_space=pl.ANY),
            pl.BlockSpec(memory_space=pl.ANY),
            pl.BlockSpec(memory_space=pl.ANY),
            pl.BlockSpec((1, 64), lambda i, dat, tag: (0, 0)),
            pl.BlockSpec(memory_space=pl.ANY),
            pl.BlockSpec((1, 50), lambda i, dat, tag: (0, 0)),
        ],
        out_specs=pl.BlockSpec((1, 1), lambda i, dat, tag: (0, 0)),
        scratch_shapes=[
            pltpu.VMEM((8, 128), jnp.float32),
            pltpu.VMEM((128, 128), jnp.float32),
            pltpu.VMEM((1, 128), jnp.float32),
            pltpu.VMEM((_N_LAYERS, 128, 128), jnp.float32),
            pltpu.VMEM((_N_LAYERS, 128), jnp.float32),
            pltpu.VMEM((128, 64), jnp.float32),
            pltpu.VMEM((64, 50), jnp.float32),
            pltpu.SemaphoreType.DMA,
        ],
    )

    res = pl.pallas_call(
        _body,
        grid_spec=grid_spec,
        out_shape=jax.ShapeDtypeStruct((1, 1), jnp.float32),
    )(
        batch_datas, batch_tags, emb_table,
        W_start, b_start.reshape(1, 128), Ws, bs,
        W_end, b_end.reshape(1, 64), W_fc, b_fc.reshape(1, 50),
    )
    return res[0, 0]


# speedup vs baseline: 3.0575x; 3.0575x over previous
import jax
import jax.numpy as jnp
import numpy as np
from jax.experimental import pallas as pl
from jax.experimental.pallas import tpu as pltpu

_N_LAYERS = 4
_BAND = _N_LAYERS + 3


def _coeffs():
    es = [1.0] + [float(l + 3) for l in range(_N_LAYERS)] + [float(_N_LAYERS + 3)]
    out = []
    for e in es:
        dinv = np.float32(np.float32(e + 1.0) ** np.float32(-0.5))
        alpha = np.float32(np.float32(dinv * np.float32(e)) * dinv)
        beta = np.float32(dinv * dinv)
        out.append((alpha, beta))
    return out


_COEFFS = _coeffs()


def _body(datas_hbm, emb_hbm, w0_hbm, b0_hbm, ws_hbm, bs_hbm,
          wet_hbm, be_ref, wfct_ref, bfc_ref, tag_ref, out_ref,
          x_scr, tok_v, w0_s, b0_s, ws_s, bs_s, wet_s, sem_t, sem):
    nb, seq = datas_hbm.shape
    ncols = seq % 128
    tile0 = seq - ncols
    tok_copy = pltpu.make_async_copy(
        datas_hbm.at[:, pl.ds(tile0, ncols)], tok_v, sem_t)
    tok_copy.start()
    wcopies = [
        pltpu.make_async_copy(w0_hbm, w0_s, sem),
        pltpu.make_async_copy(b0_hbm, b0_s, sem),
        pltpu.make_async_copy(ws_hbm, ws_s, sem),
        pltpu.make_async_copy(bs_hbm, bs_s, sem),
        pltpu.make_async_copy(wet_hbm, wet_s, sem),
    ]
    for c in wcopies:
        c.start()
    x_scr[pl.ds(_BAND, 1), :] = jnp.zeros((1, 128), jnp.float32)
    tok_copy.wait()
    gcopies = [
        pltpu.make_async_copy(
            emb_hbm.at[pl.ds(tok_v[nb - 1, ncols - _BAND + j], 1), :],
            x_scr.at[pl.ds(j, 1), :], sem)
        for j in range(_BAND)
    ]
    for c in gcopies:
        c.start()
    for c in wcopies + gcopies:
        c.wait()

    def conv(x, w, b, k, relu):
        a, bt = _COEFFS[k]
        y = jnp.dot(x, w, preferred_element_type=jnp.float32)
        shifted = jnp.concatenate([jnp.zeros_like(y[:1]), y[:-1]], axis=0)
        y = a * shifted + bt * y + b
        return jnp.maximum(y, 0.0) if relu else y

    x = x_scr[...]
    x = conv(x, w0_s[...], b0_s[...], 0, True)
    for l in range(_N_LAYERS):
        x = conv(x, ws_s[l], bs_s[l:l + 1, :], l + 1, True)
    yt = jax.lax.dot_general(x, wet_s[...], (((1,), (1,)), ((), ())),
                             preferred_element_type=jnp.float32)
    ae, be_c = _COEFFS[_N_LAYERS + 1]
    sh = jnp.concatenate([jnp.zeros_like(yt[:1]), yt[:-1]], axis=0)
    x = ae * sh + be_c * yt + be_ref[...]
    xrow = x[_BAND - 1:_BAND, :]
    row = jax.lax.dot_general(xrow, wfct_ref[...], (((1,), (1,)), ((), ())),
                              preferred_element_type=jnp.float32)
    row = row + bfc_ref[...]
    m = jnp.max(row, axis=1, keepdims=True)
    lse = m + jnp.log(jnp.sum(jnp.exp(row - m), axis=1, keepdims=True))
    lane = jax.lax.broadcasted_iota(jnp.int32, row.shape, 1)
    picked = jnp.sum(jnp.where(lane == tag_ref[0], row, 0.0), axis=1,
                     keepdims=True)
    out_ref[...] = lse - picked


def kernel(batch_datas, batch_tags, emb_table, W_start, b_start, Ws, bs,
           W_end, b_end, W_fc, b_fc):

    grid_spec = pltpu.PrefetchScalarGridSpec(
        num_scalar_prefetch=0,
        grid=(1,),
        in_specs=[
            pl.BlockSpec(memory_space=pl.ANY),
            pl.BlockSpec(memory_space=pl.ANY),
            pl.BlockSpec(memory_space=pl.ANY),
            pl.BlockSpec(memory_space=pl.ANY),
            pl.BlockSpec(memory_space=pl.ANY),
            pl.BlockSpec(memory_space=pl.ANY),
            pl.BlockSpec(memory_space=pl.ANY),
            pl.BlockSpec((1, 64), lambda i: (0, 0)),
            pl.BlockSpec((50, 64), lambda i: (0, 0)),
            pl.BlockSpec((1, 50), lambda i: (0, 0)),
            pl.BlockSpec(memory_space=pltpu.SMEM),
        ],
        out_specs=pl.BlockSpec((1, 1), lambda i: (0, 0)),
        scratch_shapes=[
            pltpu.VMEM((8, 128), jnp.float32),
            pltpu.VMEM((2, 80), jnp.int32),
            pltpu.VMEM((128, 128), jnp.float32),
            pltpu.VMEM((1, 128), jnp.float32),
            pltpu.VMEM((_N_LAYERS, 128, 128), jnp.float32),
            pltpu.VMEM((_N_LAYERS, 128), jnp.float32),
            pltpu.VMEM((64, 128), jnp.float32),
            pltpu.SemaphoreType.DMA,
            pltpu.SemaphoreType.DMA,
        ],
    )

    res = pl.pallas_call(
        _body,
        grid_spec=grid_spec,
        out_shape=jax.ShapeDtypeStruct((1, 1), jnp.float32),
    )(
        batch_datas, emb_table,
        W_start, b_start.reshape(1, 128), Ws, bs,
        W_end.T, b_end.reshape(1, 64), W_fc.T, b_fc.reshape(1, 50), batch_tags,
    )
    return res[0, 0]


# speedup vs baseline: 3.0597x; 1.0007x over previous
import jax
import jax.numpy as jnp
import numpy as np
from jax.experimental import pallas as pl
from jax.experimental.pallas import tpu as pltpu

_N_LAYERS = 4
_BAND = _N_LAYERS + 3


def _coeffs():
    es = [1.0] + [float(l + 3) for l in range(_N_LAYERS)] + [float(_N_LAYERS + 3)]
    out = []
    for e in es:
        dinv = np.float32(np.float32(e + 1.0) ** np.float32(-0.5))
        alpha = np.float32(np.float32(dinv * np.float32(e)) * dinv)
        beta = np.float32(dinv * dinv)
        out.append((alpha, beta))
    return out


_COEFFS = _coeffs()


def _body(datas_hbm, emb_hbm, w0_hbm, b0_hbm, ws_hbm, bs_hbm,
          wet_hbm, be_ref, wfct_ref, bfc_ref, tag_ref, out_ref,
          x_scr, tok_v, w0_s, b0_s, ws_s, bs_s, wet_s, sem_t, sem):
    nb, seq = datas_hbm.shape
    ncols = seq % 128
    tile0 = seq - ncols
    tok_copy = pltpu.make_async_copy(
        datas_hbm.at[:, pl.ds(tile0, ncols)], tok_v, sem_t)
    tok_copy.start()
    wcopies = [
        pltpu.make_async_copy(w0_hbm, w0_s, sem),
        pltpu.make_async_copy(b0_hbm, b0_s, sem),
        pltpu.make_async_copy(ws_hbm, ws_s, sem),
        pltpu.make_async_copy(bs_hbm, bs_s, sem),
        pltpu.make_async_copy(wet_hbm, wet_s, sem),
    ]
    for c in wcopies:
        c.start()
    x_scr[pl.ds(_BAND, 1), :] = jnp.zeros((1, 128), jnp.float32)
    tok_copy.wait()
    gcopies = [
        pltpu.make_async_copy(
            emb_hbm.at[pl.ds(tok_v[nb - 1, ncols - _BAND + j], 1), :],
            x_scr.at[pl.ds(j, 1), :], sem)
        for j in range(_BAND)
    ]
    for c in gcopies:
        c.start()
    for c in wcopies + gcopies:
        c.wait()

    def conv(x, w, b, k, relu):
        a, bt = _COEFFS[k]
        y = jnp.dot(x, w, preferred_element_type=jnp.float32)
        shifted = jnp.concatenate([jnp.zeros_like(y[:1]), y[:-1]], axis=0)
        y = a * shifted + bt * y + b
        return jnp.maximum(y, 0.0) if relu else y

    x = x_scr[...]
    x = conv(x, w0_s[...], b0_s[...], 0, True)
    for l in range(_N_LAYERS):
        x = conv(x, ws_s[l], bs_s[l:l + 1, :], l + 1, True)
    yt = jax.lax.dot_general(x, wet_s[...], (((1,), (1,)), ((), ())),
                             preferred_element_type=jnp.float32)
    ae, be_c = _COEFFS[_N_LAYERS + 1]
    sh = jnp.concatenate([jnp.zeros_like(yt[:1]), yt[:-1]], axis=0)
    x = ae * sh + be_c * yt + be_ref[...]
    xrow = x[_BAND - 1:_BAND, :]
    row = jax.lax.dot_general(xrow, wfct_ref[...], (((1,), (1,)), ((), ())),
                              preferred_element_type=jnp.float32)
    row = row + bfc_ref[...]
    m = jnp.max(row, axis=1, keepdims=True)
    lse = m + jnp.log(jnp.sum(jnp.exp(row - m), axis=1, keepdims=True))
    lane = jax.lax.broadcasted_iota(jnp.int32, row.shape, 1)
    tag = jnp.broadcast_to(tag_ref[...], row.shape)
    picked = jnp.sum(jnp.where(lane == tag, row, 0.0), axis=1,
                     keepdims=True)
    out_ref[...] = lse - picked


def kernel(batch_datas, batch_tags, emb_table, W_start, b_start, Ws, bs,
           W_end, b_end, W_fc, b_fc):

    grid_spec = pltpu.PrefetchScalarGridSpec(
        num_scalar_prefetch=0,
        grid=(1,),
        in_specs=[
            pl.BlockSpec(memory_space=pl.ANY),
            pl.BlockSpec(memory_space=pl.ANY),
            pl.BlockSpec(memory_space=pl.ANY),
            pl.BlockSpec(memory_space=pl.ANY),
            pl.BlockSpec(memory_space=pl.ANY),
            pl.BlockSpec(memory_space=pl.ANY),
            pl.BlockSpec(memory_space=pl.ANY),
            pl.BlockSpec((1, 64), lambda i: (0, 0)),
            pl.BlockSpec((50, 64), lambda i: (0, 0)),
            pl.BlockSpec((1, 50), lambda i: (0, 0)),
            pl.BlockSpec((1, 1), lambda i: (0, 0)),
        ],
        out_specs=pl.BlockSpec((1, 1), lambda i: (0, 0)),
        scratch_shapes=[
            pltpu.VMEM((8, 128), jnp.float32),
            pltpu.VMEM((2, 80), jnp.int32),
            pltpu.VMEM((128, 128), jnp.float32),
            pltpu.VMEM((1, 128), jnp.float32),
            pltpu.VMEM((_N_LAYERS, 128, 128), jnp.float32),
            pltpu.VMEM((_N_LAYERS, 128), jnp.float32),
            pltpu.VMEM((64, 128), jnp.float32),
            pltpu.SemaphoreType.DMA,
            pltpu.SemaphoreType.DMA,
        ],
    )

    res = pl.pallas_call(
        _body,
        grid_spec=grid_spec,
        out_shape=jax.ShapeDtypeStruct((1, 1), jnp.float32),
    )(
        batch_datas, emb_table,
        W_start, b_start.reshape(1, 128), Ws, bs,
        W_end.T, b_end.reshape(1, 64), W_fc.T, b_fc.reshape(1, 50), batch_tags.reshape(1, 1),
    )
    return res[0, 0]
